# Initial kernel scaffold; baseline (speedup 1.0000x reference)
import jax
import jax.numpy as jnp
from jax.experimental import pallas as pl


def kernel(grid_nodes_features, params, edge_index):
    raise NotImplementedError("write your pallas kernel here")


# speedup vs baseline: 3.3388x; 3.3388x over previous
import functools

import jax
import jax.numpy as jnp
from jax import lax
from jax.experimental import pallas as pl
from jax.experimental.pallas import tpu as pltpu
from jax.experimental.pallas import tpu_sc as plsc

D = 128
NG = 50000
NM = 10000
E = 320000

_NC = 2
_NS = 16
_NW = _NC * _NS
_EP = E // _NW
_C = 80
_NCH = _EP // _C
_ZR = 624
_ZT = NM - _ZR * (_NS - 1) - _ZR


def _ln(y, g, b):
    mu = jnp.mean(y, axis=-1, keepdims=True)
    var = jnp.mean((y - mu) ** 2, axis=-1, keepdims=True)
    return (y - mu) * lax.rsqrt(var + 1e-5) * g + b


def _full(shape):
    return pl.BlockSpec(shape, lambda i: (0,) * len(shape))


def _rows(bm, n):
    return pl.BlockSpec((bm, n), lambda i: (i, 0))


def _grid_kernel_body(gf, geo, w1a, w1b, b1, w2, b2, g, bb,
                      w1s, nw1, nb1, nw2, nb2, ng_, nbb,
                      lat_ref, g1_ref, go_ref):
    h = jax.nn.silu(
        jnp.dot(gf[...], w1a[...], preferred_element_type=jnp.float32)
        + jnp.dot(geo[...], w1b[...], preferred_element_type=jnp.float32)
        + b1[...])
    lat = _ln(jnp.dot(h, w2[...], preferred_element_type=jnp.float32) + b2[...],
              g[...], bb[...])
    lat_ref[...] = lat
    g1_ref[...] = jnp.dot(lat, w1s[...], preferred_element_type=jnp.float32)
    gh = jax.nn.silu(jnp.dot(lat, nw1[...], preferred_element_type=jnp.float32)
                     + nb1[...])
    go = _ln(jnp.dot(gh, nw2[...], preferred_element_type=jnp.float32) + nb2[...],
             ng_[...], nbb[...])
    go_ref[...] = go + lat


def _grid_stage(gfeat, geo, ge, w1s, gn):
    bm = 2000
    grid = (NG // bm,)
    r = lambda a: a.reshape(1, -1)
    out = pl.pallas_call(
        _grid_kernel_body,
        grid=grid,
        in_specs=[
            _rows(bm, 96), _rows(bm, 32),
            _full((96, D)), _full((32, D)), _full((1, D)),
            _full((D, D)), _full((1, D)), _full((1, D)), _full((1, D)),
            _full((D, D)),
            _full((D, D)), _full((1, D)), _full((D, D)), _full((1, D)),
            _full((1, D)), _full((1, D)),
        ],
        out_specs=[_rows(bm, D), _rows(bm, D), _rows(bm, D)],
        out_shape=[jax.ShapeDtypeStruct((NG, D), jnp.float32)] * 3,
    )(gfeat, geo,
      ge["W1"][:96], ge["W1"][96:], r(ge["b1"]), ge["W2"], r(ge["b2"]),
      r(ge["ln_g"]), r(ge["ln_b"]),
      w1s,
      gn["W1"], r(gn["b1"]), gn["W2"], r(gn["b2"]), r(gn["ln_g"]), r(gn["ln_b"]))
    return out


def _embed_body(x, w1, b1, w2, b2, g, bb, out_ref):
    h = jax.nn.silu(jnp.dot(x[...], w1[...], preferred_element_type=jnp.float32)
                    + b1[...])
    out_ref[...] = _ln(
        jnp.dot(h, w2[...], preferred_element_type=jnp.float32) + b2[...],
        g[...], bb[...])


def _embed_stage(x, p, bm):
    n, k = x.shape
    r = lambda a: a.reshape(1, -1)
    return pl.pallas_call(
        _embed_body,
        grid=(n // bm,),
        in_specs=[_rows(bm, k), _full((k, D)), _full((1, D)), _full((D, D)),
                  _full((1, D)), _full((1, D)), _full((1, D))],
        out_specs=_rows(bm, D),
        out_shape=jax.ShapeDtypeStruct((n, D), jnp.float32),
    )(x, p["W1"], r(p["b1"]), p["W2"], r(p["b2"]), r(p["ln_g"]), r(p["ln_b"]))


def _proj_body(x, w, b, out_ref):
    out_ref[...] = (jnp.dot(x[...], w[...], preferred_element_type=jnp.float32)
                    + b[...])


def _proj_stage(x, w, b, bm):
    n = x.shape[0]
    return pl.pallas_call(
        _proj_body,
        grid=(n // bm,),
        in_specs=[_rows(bm, D), _full((D, D)), _full((1, D))],
        out_specs=_rows(bm, D),
        out_shape=jax.ShapeDtypeStruct((n, D), jnp.float32),
    )(x, w, b.reshape(1, -1))


def _edge_mlp_body(el, sf, df, w1, w2, b2, g, bb, out_ref):
    h = jax.nn.silu(jnp.dot(el[...], w1[...], preferred_element_type=jnp.float32)
                    + sf[...] + df[...])
    y = _ln(jnp.dot(h, w2[...], preferred_element_type=jnp.float32) + b2[...],
            g[...], bb[...])
    out_ref[...] = el[...] + y


def _edge_mlp_stage(el, sf, df, w1e, p):
    bm = 2000
    r = lambda a: a.reshape(1, -1)
    return pl.pallas_call(
        _edge_mlp_body,
        grid=(E // bm,),
        in_specs=[_rows(bm, D), _rows(bm, D), _rows(bm, D),
                  _full((D, D)), _full((D, D)), _full((1, D)), _full((1, D)),
                  _full((1, D))],
        out_specs=_rows(bm, D),
        out_shape=jax.ShapeDtypeStruct((E, D), jnp.float32),
    )(el, sf, df, w1e, p["W2"], r(p["b2"]), r(p["ln_g"]), r(p["ln_b"]))


def _node_mlp_body(ml, a0, a1, w1m, w1a, b1, w2, b2, g, bb, out_ref):
    agg = a0[...] + a1[...]
    h = jax.nn.silu(jnp.dot(ml[...], w1m[...], preferred_element_type=jnp.float32)
                    + jnp.dot(agg, w1a[...], preferred_element_type=jnp.float32)
                    + b1[...])
    y = _ln(jnp.dot(h, w2[...], preferred_element_type=jnp.float32) + b2[...],
            g[...], bb[...])
    out_ref[...] = ml[...] + y


def _node_mlp_stage(ml, agg2, p):
    bm = 2000
    r = lambda a: a.reshape(1, -1)
    return pl.pallas_call(
        _node_mlp_body,
        grid=(NM // bm,),
        in_specs=[_rows(bm, D), _rows(bm, D), _rows(bm, D),
                  _full((D, D)), _full((D, D)), _full((1, D)),
                  _full((D, D)), _full((1, D)), _full((1, D)), _full((1, D))],
        out_specs=_rows(bm, D),
        out_shape=jax.ShapeDtypeStruct((NM, D), jnp.float32),
    )(ml, agg2[0], agg2[1],
      p["W1"][:D], p["W1"][D:], r(p["b1"]), p["W2"], r(p["b2"]),
      r(p["ln_g"]), r(p["ln_b"]))


def _sc_mesh():
    return plsc.VectorSubcoreMesh(core_axis_name="c", subcore_axis_name="s",
                                  num_cores=_NC, num_subcores=_NS)


def _sc_gather(g1, m1, src, dst):
    @functools.partial(
        pl.kernel,
        out_type=(jax.ShapeDtypeStruct((E, D), jnp.float32),
                  jax.ShapeDtypeStruct((E, D), jnp.float32)),
        mesh=_sc_mesh(),
        scratch_types=[
            pltpu.VMEM((_C,), jnp.int32),
            pltpu.VMEM((_C,), jnp.int32),
            pltpu.VMEM((_C, D), jnp.float32),
            pltpu.VMEM((_C, D), jnp.float32),
            pltpu.SemaphoreType.DMA,
            pltpu.SemaphoreType.DMA,
        ],
    )
    def k(g1_hbm, m1_hbm, src_hbm, dst_hbm, sf_hbm, df_hbm,
          sidx, didx, srows, drows, sem1, sem2):
        wid = lax.axis_index("s") * _NC + lax.axis_index("c")
        base = wid * _EP

        def body(i, carry):
            off = base + i * _C
            pltpu.sync_copy(src_hbm.at[pl.ds(off, _C)], sidx)
            pltpu.sync_copy(dst_hbm.at[pl.ds(off, _C)], didx)
            cp1 = pltpu.async_copy(g1_hbm.at[sidx], srows, sem1)
            cp2 = pltpu.async_copy(m1_hbm.at[didx], drows, sem2)
            cp1.wait()
            cp2.wait()
            pltpu.sync_copy(srows, sf_hbm.at[pl.ds(off, _C)])
            pltpu.sync_copy(drows, df_hbm.at[pl.ds(off, _C)])
            return carry

        lax.fori_loop(0, _NCH, body, 0)

    return k(g1, m1, src, dst)


def _sc_scatter(edge2, dst, zeros_nm):
    @functools.partial(
        pl.kernel,
        out_type=jax.ShapeDtypeStruct((_NC, NM, D), jnp.float32),
        mesh=_sc_mesh(),
        scratch_types=[
            pltpu.VMEM((_C,), jnp.int32),
            pltpu.VMEM((_C, D), jnp.float32),
            pltpu.VMEM_SHARED((NM, D), jnp.float32),
        ],
    )
    def k(e_hbm, dst_hbm, z_hbm, out_hbm, idx, rows, acc):
        c = lax.axis_index("c")
        s = lax.axis_index("s")
        wid = s * _NC + c
        pltpu.sync_copy(z_hbm.at[pl.ds(s * _ZR, _ZR)],
                        acc.at[pl.ds(s * _ZR, _ZR)])
        @pl.when(s == _NS - 1)
        def _():
            pltpu.sync_copy(z_hbm.at[pl.ds(_NS * _ZR, _ZT)],
                            acc.at[pl.ds(_NS * _ZR, _ZT)])
        plsc.subcore_barrier()
        base = wid * _EP

        def body(i, carry):
            off = base + i * _C
            pltpu.sync_copy(dst_hbm.at[pl.ds(off, _C)], idx)
            pltpu.sync_copy(e_hbm.at[pl.ds(off, _C)], rows)
            pltpu.sync_copy(rows, acc.at[idx], add=True)
            return carry

        lax.fori_loop(0, _NCH, body, 0)
        plsc.subcore_barrier()
        pltpu.sync_copy(acc.at[pl.ds(s * _ZR, _ZR)],
                        out_hbm.at[c, pl.ds(s * _ZR, _ZR)])
        @pl.when(s == _NS - 1)
        def _():
            pltpu.sync_copy(acc.at[pl.ds(_NS * _ZR, _ZT)],
                            out_hbm.at[c, pl.ds(_NS * _ZR, _ZT)])

    return k(edge2, dst, zeros_nm)


def kernel(grid_nodes_features, params, edge_index):
    p = params
    gfeat = grid_nodes_features[0]
    src = edge_index[0]
    dst = edge_index[1]
    w1 = p["edge_mlp"]["W1"]
    w1e, w1s, w1d = w1[:D], w1[D:2 * D], w1[2 * D:]

    grid_lat, g1, grid_out = _grid_stage(gfeat, p["grid_geo"], p["grid_embed"],
                                         w1s, p["grid_node_mlp"])
    mesh_lat = _embed_stage(p["mesh_geo"], p["mesh_embed"], 2000)
    m1 = _proj_stage(mesh_lat, w1d, p["edge_mlp"]["b1"], 2000)
    edge_lat = _embed_stage(p["edge_feats"], p["edge_embed"], 2000)

    sf, df = _sc_gather(g1, m1, src, dst)
    edge2 = _edge_mlp_stage(edge_lat, sf, df, w1e, p["edge_mlp"])
    agg2 = _sc_scatter(edge2, dst, jnp.zeros((NM, D), jnp.float32))
    mesh_out = _node_mlp_stage(mesh_lat, agg2, p["node_mlp"])

    return (grid_out[None], mesh_out[None])
